# Initial kernel scaffold; baseline (speedup 1.0000x reference)
import jax
import jax.numpy as jnp
from jax.experimental import pallas as pl


def kernel(s):
    raise NotImplementedError("write your pallas kernel here")


# speedup vs baseline: 7.2851x; 7.2851x over previous
import jax
import jax.numpy as jnp
from jax.experimental import pallas as pl
from jax.experimental.pallas import tpu as pltpu

_MAX_ITER = 10
_EPSILON = 1e-4


def _sinkhorn_body(s_ref, o_ref):
    s = s_ref[0]
    nonzero_mask = (s != 0.0).astype(s.dtype)
    for i in range(_MAX_ITER):
        if i % 2 == 0:
            col_sum = jnp.sum(s, axis=0, keepdims=True)
            s = s * (1.0 / col_sum) * nonzero_mask
        else:
            row_sum = jnp.sum(s, axis=1, keepdims=True)
            s = (1.0 / (row_sum + _EPSILON)) * s * nonzero_mask
    o_ref[0] = s


def kernel(s):
    b, n1, n2 = s.shape
    return pl.pallas_call(
        _sinkhorn_body,
        grid=(b,),
        in_specs=[pl.BlockSpec((1, n1, n2), lambda i: (i, 0, 0))],
        out_specs=pl.BlockSpec((1, n1, n2), lambda i: (i, 0, 0)),
        out_shape=jax.ShapeDtypeStruct(s.shape, s.dtype),
        compiler_params=pltpu.CompilerParams(
            dimension_semantics=("parallel",),
        ),
    )(s)
